# baseline (speedup 1.0000x reference)
import functools

import jax
import jax.numpy as jnp
from jax import lax
from jax.experimental import pallas as pl
from jax.experimental.pallas import tpu as pltpu
from jax.experimental.pallas import tpu_sc as plsc

N = 10000
E = 320000
D = 128
NC = 2
NS = 16
NW = NC * NS
CH = 80
SB = 25
NSB = E // NW // CH // SB
ROWS_PER_TILE = N // NS
ES = 8

_R = 2000
_GRID = N // _R


def _tc1_body(x_ref, w_ref, asd_ref, hp_ref, av_ref, amax_ref):
    h = jnp.dot(x_ref[...], w_ref[...], preferred_element_type=jnp.float32)
    av = jnp.dot(h, asd_ref[...], preferred_element_type=jnp.float32)
    hp_ref[...] = h
    av_ref[...] = av
    m = jnp.max(av[:, 0:1]).reshape(1, 1)
    i = pl.program_id(0)

    @pl.when(i == 0)
    def _():
        amax_ref[...] = m

    @pl.when(i > 0)
    def _():
        amax_ref[...] = jnp.maximum(amax_ref[...], m)


def _tc_dense1(x, W1, asd1):
    return pl.pallas_call(
        _tc1_body,
        grid=(_GRID,),
        in_specs=[
            pl.BlockSpec((_R, D), lambda i: (i, 0)),
            pl.BlockSpec((D, D), lambda i: (0, 0)),
            pl.BlockSpec((D, 2), lambda i: (0, 0)),
        ],
        out_specs=[
            pl.BlockSpec((_R, D), lambda i: (i, 0)),
            pl.BlockSpec((_R, 2), lambda i: (i, 0)),
            pl.BlockSpec((1, 1), lambda i: (0, 0)),
        ],
        out_shape=[
            jax.ShapeDtypeStruct((N, D), jnp.float32),
            jax.ShapeDtypeStruct((N, 2), jnp.float32),
            jax.ShapeDtypeStruct((1, 1), jnp.float32),
        ],
    )(x, W1, asd1)


def _tc2_body(acc_ref, es_ref, b_ref, w_ref, asd_ref, hp_ref, av_ref,
              amax_ref):
    a = acc_ref[0] + acc_ref[1]
    es = es_ref[0, :, 0:1] + es_ref[1, :, 0:1]
    o1 = a / (es + 1e-16) + b_ref[...]
    g = jnp.where(o1 > 0, o1, jnp.exp(jnp.minimum(o1, 0.0)) - 1.0)
    h = jnp.dot(g, w_ref[...], preferred_element_type=jnp.float32)
    av = jnp.dot(h, asd_ref[...], preferred_element_type=jnp.float32)
    hp_ref[...] = h
    av_ref[...] = av
    m = jnp.max(av[:, 0:1]).reshape(1, 1)
    i = pl.program_id(0)

    @pl.when(i == 0)
    def _():
        amax_ref[...] = m

    @pl.when(i > 0)
    def _():
        amax_ref[...] = jnp.maximum(amax_ref[...], m)


def _tc_dense2(acc, es, b1, W2, asd2):
    return pl.pallas_call(
        _tc2_body,
        grid=(_GRID,),
        in_specs=[
            pl.BlockSpec((NC, _R, D), lambda i: (0, i, 0)),
            pl.BlockSpec((NC, _R, ES), lambda i: (0, i, 0)),
            pl.BlockSpec((1, D), lambda i: (0, 0)),
            pl.BlockSpec((D, D), lambda i: (0, 0)),
            pl.BlockSpec((D, 2), lambda i: (0, 0)),
        ],
        out_specs=[
            pl.BlockSpec((_R, D), lambda i: (i, 0)),
            pl.BlockSpec((_R, 2), lambda i: (i, 0)),
            pl.BlockSpec((1, 1), lambda i: (0, 0)),
        ],
        out_shape=[
            jax.ShapeDtypeStruct((N, D), jnp.float32),
            jax.ShapeDtypeStruct((N, 2), jnp.float32),
            jax.ShapeDtypeStruct((1, 1), jnp.float32),
        ],
    )(acc, es, b1, W2, asd2)


def _tc3_body(acc_ref, es_ref, b_ref, out_ref):
    a = acc_ref[0] + acc_ref[1]
    es = es_ref[0, :, 0:1] + es_ref[1, :, 0:1]
    out_ref[...] = a / (es + 1e-16) + b_ref[...]


def _tc_final(acc, es, b2):
    return pl.pallas_call(
        _tc3_body,
        grid=(_GRID,),
        in_specs=[
            pl.BlockSpec((NC, _R, D), lambda i: (0, i, 0)),
            pl.BlockSpec((NC, _R, ES), lambda i: (0, i, 0)),
            pl.BlockSpec((1, D), lambda i: (0, 0)),
        ],
        out_specs=pl.BlockSpec((_R, D), lambda i: (i, 0)),
        out_shape=jax.ShapeDtypeStruct((N, D), jnp.float32),
    )(acc, es, b2)


def _sc_edge_body(hp, srcm, dstm, as_h, ad_h, amax_h, out_acc, out_es,
                  src_sb, dst_sb, as_v, ad_v, amax_v, rows2, tbuf2,
                  acc_sh, esum_sh, gsem, ssem, esem):
    cid = lax.axis_index("c")
    sid = lax.axis_index("s")
    wid = sid * NC + cid
    row0 = sid * ROWS_PER_TILE

    z16 = jnp.zeros((16,), jnp.float32)
    for r in range(2 * CH):
        for j in range(D // 16):
            rows2[r, pl.ds(j * 16, 16)] = z16
    for q in range(ROWS_PER_TILE // CH):
        pltpu.sync_copy(rows2.at[pl.ds(0, CH)],
                        acc_sh.at[pl.ds(row0 + q * CH, CH)])
    pltpu.sync_copy(rows2.at[pl.ds(0, ROWS_PER_TILE % CH)],
                    acc_sh.at[pl.ds(row0 + (ROWS_PER_TILE // CH) * CH,
                                    ROWS_PER_TILE % CH)])
    iota16 = lax.iota(jnp.int32, 16)
    for m in range(2 * CH * ES // 16):
        b = iota16 + m * 16
        plsc.store_scatter(tbuf2, [b >> 3, b & 7], z16)
    for q in range(ROWS_PER_TILE // CH):
        pltpu.sync_copy(tbuf2.at[pl.ds(0, CH)],
                        esum_sh.at[pl.ds(row0 + q * CH, CH)])
    pltpu.sync_copy(tbuf2.at[pl.ds(0, ROWS_PER_TILE % CH)],
                    esum_sh.at[pl.ds(row0 + (ROWS_PER_TILE // CH) * CH,
                                     ROWS_PER_TILE % CH)])

    pltpu.sync_copy(as_h, as_v)
    pltpu.sync_copy(ad_h, ad_v)
    pltpu.sync_copy(amax_h, amax_v)
    amax = amax_v[...]

    plsc.subcore_barrier()

    lanes0 = jnp.zeros((16,), jnp.int32)

    def gath_start(g, boff):
        pltpu.async_copy(hp.at[src_sb.at[g]],
                         rows2.at[pl.ds(boff, CH)], gsem)

    def gath_wait(g, boff):
        pltpu.make_async_copy(hp.at[src_sb.at[g]],
                              rows2.at[pl.ds(boff, CH)], gsem).wait()

    def scat_wait(g, boff):
        for k in range(CH // 16):
            pltpu.make_async_copy(
                rows2.at[pl.ds(boff + k * 16, 16)],
                acc_sh.at[dst_sb.at[g, pl.ds(k * 16, 16)]], ssem).wait()
        pltpu.make_async_copy(tbuf2.at[pl.ds(boff, CH)],
                              esum_sh.at[dst_sb.at[g]], esem).wait()

    def chunk(g, carry):
        boff = (g & 1) * CH
        boff2 = CH - boff
        gath_wait(g, boff)
        @pl.when(g >= 1)
        def _():
            scat_wait(g - 1, boff2)

        @pl.when(g + 1 < SB)
        def _():
            gath_start(g + 1, boff2)

        for k in range(CH // 16):
            sv = src_sb[g, pl.ds(k * 16, 16)]
            dv = dst_sb[g, pl.ds(k * 16, 16)]
            a1 = plsc.load_gather(as_v, [sv])
            a2 = plsc.load_gather(ad_v, [dv])
            e = a1 + a2
            e = jnp.where(e >= 0, e, 0.2 * e)
            c = amax + a2
            c = jnp.where(c >= 0, c, 0.2 * c)
            t = jnp.exp(e - c)
            plsc.store_scatter(tbuf2, [iota16 + (k * 16 + boff), lanes0], t)
            for i in range(16):
                ti = jnp.broadcast_to(t[i], (16,))
                r = k * 16 + i
                for j in range(D // 16):
                    rows2[boff + r, pl.ds(j * 16, 16)] = (
                        rows2[boff + r, pl.ds(j * 16, 16)] * ti)
            pltpu.async_copy(
                rows2.at[pl.ds(boff + k * 16, 16)],
                acc_sh.at[dst_sb.at[g, pl.ds(k * 16, 16)]], ssem, add=True)
        pltpu.async_copy(tbuf2.at[pl.ds(boff, CH)],
                         esum_sh.at[dst_sb.at[g]], esem, add=True)
        return carry

    def super_chunk(s, carry):
        pltpu.sync_copy(srcm.at[wid].at[s], src_sb)
        pltpu.sync_copy(dstm.at[wid].at[s], dst_sb)
        gath_start(0, 0)
        lax.fori_loop(0, SB, chunk, 0)
        scat_wait(SB - 1, ((SB - 1) & 1) * CH)
        return carry

    lax.fori_loop(0, NSB, super_chunk, 0)

    plsc.subcore_barrier()

    pltpu.sync_copy(acc_sh.at[pl.ds(row0, ROWS_PER_TILE)],
                    out_acc.at[cid].at[pl.ds(row0, ROWS_PER_TILE)])
    pltpu.sync_copy(esum_sh.at[pl.ds(row0, ROWS_PER_TILE)],
                    out_es.at[cid].at[pl.ds(row0, ROWS_PER_TILE)])


_sc_edge = functools.partial(
    pl.kernel,
    out_type=[
        jax.ShapeDtypeStruct((NC, N, D), jnp.float32),
        jax.ShapeDtypeStruct((NC, N, ES), jnp.float32),
    ],
    mesh=plsc.VectorSubcoreMesh(core_axis_name="c", subcore_axis_name="s"),
    compiler_params=pltpu.CompilerParams(
        use_tc_tiling_on_sc=False, needs_layout_passes=False),
    scratch_types=[
        pltpu.VMEM((SB, CH), jnp.int32),
        pltpu.VMEM((SB, CH), jnp.int32),
        pltpu.VMEM((N,), jnp.float32),
        pltpu.VMEM((N,), jnp.float32),
        pltpu.VMEM((16,), jnp.float32),
        pltpu.VMEM((2 * CH, D), jnp.float32),
        pltpu.VMEM((2 * CH, ES), jnp.float32),
        pltpu.VMEM_SHARED((N, D), jnp.float32),
        pltpu.VMEM_SHARED((N, ES), jnp.float32),
        pltpu.SemaphoreType.DMA,
        pltpu.SemaphoreType.DMA,
        pltpu.SemaphoreType.DMA,
    ],
)(_sc_edge_body)


def kernel(x, edge_index, W1, a_s1, a_d1, b1, W2, a_s2, a_d2, b2):
    src = edge_index[0].astype(jnp.int32).reshape(NW, NSB, SB, CH)
    dst = edge_index[1].astype(jnp.int32).reshape(NW, NSB, SB, CH)

    asd1 = jnp.concatenate([a_s1.reshape(D, 1), a_d1.reshape(D, 1)], axis=1)
    asd2 = jnp.concatenate([a_s2.reshape(D, 1), a_d2.reshape(D, 1)], axis=1)

    hp1, av1, amax1 = _tc_dense1(x, W1, asd1)
    as1 = av1[:, 0]
    ad1 = av1[:, 1]
    amax1v = jnp.broadcast_to(amax1.reshape(()), (16,))
    acc1, es1 = _sc_edge(hp1, src, dst, as1, ad1, amax1v)

    hp2, av2, amax2 = _tc_dense2(acc1, es1, b1.reshape(1, D), W2, asd2)
    as2 = av2[:, 0]
    ad2 = av2[:, 1]
    amax2v = jnp.broadcast_to(amax2.reshape(()), (16,))
    acc2, es2 = _sc_edge(hp2, src, dst, as2, ad2, amax2v)

    return _tc_final(acc2, es2, b2.reshape(1, D))


# speedup vs baseline: 1.0142x; 1.0142x over previous
import functools

import jax
import jax.numpy as jnp
from jax import lax
from jax.experimental import pallas as pl
from jax.experimental.pallas import tpu as pltpu
from jax.experimental.pallas import tpu_sc as plsc

N = 10000
E = 320000
D = 128
NC = 2
NS = 16
NW = NC * NS
CH = 80
SB = 25
NSB = E // NW // CH // SB
ROWS_PER_TILE = N // NS
ES = 8

_R = 2000
_GRID = N // _R


def _tc1_body(x_ref, w_ref, asd_ref, hp_ref, av_ref, amax_ref):
    h = jnp.dot(x_ref[...], w_ref[...], preferred_element_type=jnp.float32)
    av = jnp.dot(h, asd_ref[...], preferred_element_type=jnp.float32)
    hp_ref[...] = h
    av_ref[...] = av
    m = jnp.max(av[:, 0:1]).reshape(1, 1)
    i = pl.program_id(0)

    @pl.when(i == 0)
    def _():
        amax_ref[...] = m

    @pl.when(i > 0)
    def _():
        amax_ref[...] = jnp.maximum(amax_ref[...], m)


def _tc_dense1(x, W1, asd1):
    return pl.pallas_call(
        _tc1_body,
        grid=(_GRID,),
        in_specs=[
            pl.BlockSpec((_R, D), lambda i: (i, 0)),
            pl.BlockSpec((D, D), lambda i: (0, 0)),
            pl.BlockSpec((D, 2), lambda i: (0, 0)),
        ],
        out_specs=[
            pl.BlockSpec((_R, D), lambda i: (i, 0)),
            pl.BlockSpec((_R, 2), lambda i: (i, 0)),
            pl.BlockSpec((1, 1), lambda i: (0, 0)),
        ],
        out_shape=[
            jax.ShapeDtypeStruct((N, D), jnp.float32),
            jax.ShapeDtypeStruct((N, 2), jnp.float32),
            jax.ShapeDtypeStruct((1, 1), jnp.float32),
        ],
    )(x, W1, asd1)


def _tc2_body(acc_ref, es_ref, b_ref, w_ref, asd_ref, hp_ref, av_ref,
              amax_ref):
    a = acc_ref[0] + acc_ref[1]
    es = es_ref[0, :, 0:1] + es_ref[1, :, 0:1]
    o1 = a / (es + 1e-16) + b_ref[...]
    g = jnp.where(o1 > 0, o1, jnp.exp(jnp.minimum(o1, 0.0)) - 1.0)
    h = jnp.dot(g, w_ref[...], preferred_element_type=jnp.float32)
    av = jnp.dot(h, asd_ref[...], preferred_element_type=jnp.float32)
    hp_ref[...] = h
    av_ref[...] = av
    m = jnp.max(av[:, 0:1]).reshape(1, 1)
    i = pl.program_id(0)

    @pl.when(i == 0)
    def _():
        amax_ref[...] = m

    @pl.when(i > 0)
    def _():
        amax_ref[...] = jnp.maximum(amax_ref[...], m)


def _tc_dense2(acc, es, b1, W2, asd2):
    return pl.pallas_call(
        _tc2_body,
        grid=(_GRID,),
        in_specs=[
            pl.BlockSpec((NC, _R, D), lambda i: (0, i, 0)),
            pl.BlockSpec((NC, _R, ES), lambda i: (0, i, 0)),
            pl.BlockSpec((1, D), lambda i: (0, 0)),
            pl.BlockSpec((D, D), lambda i: (0, 0)),
            pl.BlockSpec((D, 2), lambda i: (0, 0)),
        ],
        out_specs=[
            pl.BlockSpec((_R, D), lambda i: (i, 0)),
            pl.BlockSpec((_R, 2), lambda i: (i, 0)),
            pl.BlockSpec((1, 1), lambda i: (0, 0)),
        ],
        out_shape=[
            jax.ShapeDtypeStruct((N, D), jnp.float32),
            jax.ShapeDtypeStruct((N, 2), jnp.float32),
            jax.ShapeDtypeStruct((1, 1), jnp.float32),
        ],
    )(acc, es, b1, W2, asd2)


def _tc3_body(acc_ref, es_ref, b_ref, out_ref):
    a = acc_ref[0] + acc_ref[1]
    es = es_ref[0, :, 0:1] + es_ref[1, :, 0:1]
    out_ref[...] = a / (es + 1e-16) + b_ref[...]


def _tc_final(acc, es, b2):
    return pl.pallas_call(
        _tc3_body,
        grid=(_GRID,),
        in_specs=[
            pl.BlockSpec((NC, _R, D), lambda i: (0, i, 0)),
            pl.BlockSpec((NC, _R, ES), lambda i: (0, i, 0)),
            pl.BlockSpec((1, D), lambda i: (0, 0)),
        ],
        out_specs=pl.BlockSpec((_R, D), lambda i: (i, 0)),
        out_shape=jax.ShapeDtypeStruct((N, D), jnp.float32),
    )(acc, es, b2)


def _sc_edge_body(hp, srcm, dstm, as_h, ad_h, amax_h, out_acc, out_es,
                  src_sb, dst_sb, as_v, ad_v, amax_v, rows2, tbuf2,
                  acc_sh, esum_sh, gsem, ssem, esem):
    cid = lax.axis_index("c")
    sid = lax.axis_index("s")
    wid = sid * NC + cid
    row0 = sid * ROWS_PER_TILE

    z16 = jnp.zeros((16,), jnp.float32)
    for r in range(2 * CH):
        for j in range(D // 16):
            rows2[r, pl.ds(j * 16, 16)] = z16
    for q in range(ROWS_PER_TILE // CH):
        pltpu.sync_copy(rows2.at[pl.ds(0, CH)],
                        acc_sh.at[pl.ds(row0 + q * CH, CH)])
    pltpu.sync_copy(rows2.at[pl.ds(0, ROWS_PER_TILE % CH)],
                    acc_sh.at[pl.ds(row0 + (ROWS_PER_TILE // CH) * CH,
                                    ROWS_PER_TILE % CH)])
    iota16 = lax.iota(jnp.int32, 16)
    for m in range(2 * CH * ES // 16):
        b = iota16 + m * 16
        plsc.store_scatter(tbuf2, [b >> 3, b & 7], z16)
    for q in range(ROWS_PER_TILE // CH):
        pltpu.sync_copy(tbuf2.at[pl.ds(0, CH)],
                        esum_sh.at[pl.ds(row0 + q * CH, CH)])
    pltpu.sync_copy(tbuf2.at[pl.ds(0, ROWS_PER_TILE % CH)],
                    esum_sh.at[pl.ds(row0 + (ROWS_PER_TILE // CH) * CH,
                                     ROWS_PER_TILE % CH)])

    pltpu.sync_copy(as_h, as_v)
    pltpu.sync_copy(ad_h, ad_v)
    pltpu.sync_copy(amax_h, amax_v)
    amax = amax_v[...]

    plsc.subcore_barrier()

    lanes0 = jnp.zeros((16,), jnp.int32)

    def gath_start(g, boff):
        pltpu.async_copy(hp.at[src_sb.at[g]],
                         rows2.at[pl.ds(boff, CH)], gsem)

    def gath_wait(g, boff):
        pltpu.make_async_copy(hp.at[src_sb.at[g]],
                              rows2.at[pl.ds(boff, CH)], gsem).wait()

    def scat_start(g, boff):
        pltpu.async_copy(rows2.at[pl.ds(boff, CH)],
                         acc_sh.at[dst_sb.at[g]], ssem, add=True)
        pltpu.async_copy(tbuf2.at[pl.ds(boff, CH)],
                         esum_sh.at[dst_sb.at[g]], esem, add=True)

    def scat_wait(g, boff):
        pltpu.make_async_copy(rows2.at[pl.ds(boff, CH)],
                              acc_sh.at[dst_sb.at[g]], ssem).wait()
        pltpu.make_async_copy(tbuf2.at[pl.ds(boff, CH)],
                              esum_sh.at[dst_sb.at[g]], esem).wait()

    def chunk(g, carry):
        boff = (g & 1) * CH
        boff2 = CH - boff
        gath_wait(g, boff)
        @pl.when(g >= 1)
        def _():
            scat_wait(g - 1, boff2)

        @pl.when(g + 1 < SB)
        def _():
            gath_start(g + 1, boff2)

        for k in range(CH // 16):
            sv = src_sb[g, pl.ds(k * 16, 16)]
            dv = dst_sb[g, pl.ds(k * 16, 16)]
            a1 = plsc.load_gather(as_v, [sv])
            a2 = plsc.load_gather(ad_v, [dv])
            e = a1 + a2
            e = jnp.where(e >= 0, e, 0.2 * e)
            c = amax + a2
            c = jnp.where(c >= 0, c, 0.2 * c)
            t = jnp.exp(e - c)
            plsc.store_scatter(tbuf2, [iota16 + (k * 16 + boff), lanes0], t)
            for i in range(16):
                ti = jnp.broadcast_to(t[i], (16,))
                r = k * 16 + i
                for j in range(D // 16):
                    rows2[boff + r, pl.ds(j * 16, 16)] = (
                        rows2[boff + r, pl.ds(j * 16, 16)] * ti)
        scat_start(g, boff)
        return carry

    def super_chunk(s, carry):
        pltpu.sync_copy(srcm.at[wid].at[s], src_sb)
        pltpu.sync_copy(dstm.at[wid].at[s], dst_sb)
        gath_start(0, 0)
        lax.fori_loop(0, SB, chunk, 0)
        scat_wait(SB - 1, ((SB - 1) & 1) * CH)
        return carry

    lax.fori_loop(0, NSB, super_chunk, 0)

    plsc.subcore_barrier()

    pltpu.sync_copy(acc_sh.at[pl.ds(row0, ROWS_PER_TILE)],
                    out_acc.at[cid].at[pl.ds(row0, ROWS_PER_TILE)])
    pltpu.sync_copy(esum_sh.at[pl.ds(row0, ROWS_PER_TILE)],
                    out_es.at[cid].at[pl.ds(row0, ROWS_PER_TILE)])


_sc_edge = functools.partial(
    pl.kernel,
    out_type=[
        jax.ShapeDtypeStruct((NC, N, D), jnp.float32),
        jax.ShapeDtypeStruct((NC, N, ES), jnp.float32),
    ],
    mesh=plsc.VectorSubcoreMesh(core_axis_name="c", subcore_axis_name="s"),
    compiler_params=pltpu.CompilerParams(
        use_tc_tiling_on_sc=False, needs_layout_passes=False),
    scratch_types=[
        pltpu.VMEM((SB, CH), jnp.int32),
        pltpu.VMEM((SB, CH), jnp.int32),
        pltpu.VMEM((N,), jnp.float32),
        pltpu.VMEM((N,), jnp.float32),
        pltpu.VMEM((16,), jnp.float32),
        pltpu.VMEM((2 * CH, D), jnp.float32),
        pltpu.VMEM((2 * CH, ES), jnp.float32),
        pltpu.VMEM_SHARED((N, D), jnp.float32),
        pltpu.VMEM_SHARED((N, ES), jnp.float32),
        pltpu.SemaphoreType.DMA,
        pltpu.SemaphoreType.DMA,
        pltpu.SemaphoreType.DMA,
    ],
)(_sc_edge_body)


def kernel(x, edge_index, W1, a_s1, a_d1, b1, W2, a_s2, a_d2, b2):
    src = edge_index[0].astype(jnp.int32).reshape(NW, NSB, SB, CH)
    dst = edge_index[1].astype(jnp.int32).reshape(NW, NSB, SB, CH)

    asd1 = jnp.concatenate([a_s1.reshape(D, 1), a_d1.reshape(D, 1)], axis=1)
    asd2 = jnp.concatenate([a_s2.reshape(D, 1), a_d2.reshape(D, 1)], axis=1)

    hp1, av1, amax1 = _tc_dense1(x, W1, asd1)
    as1 = av1[:, 0]
    ad1 = av1[:, 1]
    amax1v = jnp.broadcast_to(amax1.reshape(()), (16,))
    acc1, es1 = _sc_edge(hp1, src, dst, as1, ad1, amax1v)

    hp2, av2, amax2 = _tc_dense2(acc1, es1, b1.reshape(1, D), W2, asd2)
    as2 = av2[:, 0]
    ad2 = av2[:, 1]
    amax2v = jnp.broadcast_to(amax2.reshape(()), (16,))
    acc2, es2 = _sc_edge(hp2, src, dst, as2, ad2, amax2v)

    return _tc_final(acc2, es2, b2.reshape(1, D))
